# baseline (speedup 1.0000x reference)
import jax
import jax.numpy as jnp
from jax.experimental import pallas as pl


def _lookup_body(pe_ref, out_ref):
    out_ref[...] = pe_ref[...]


def kernel(x, pos_embed):
    seq_len = x.shape[0]
    d_model = pos_embed.shape[1]
    out = pl.pallas_call(
        _lookup_body,
        out_shape=jax.ShapeDtypeStruct((seq_len, d_model), pos_embed.dtype),
    )(pos_embed[:seq_len])
    return out[:, None, :]


# speedup vs baseline: 2.0708x; 2.0708x over previous
import jax
import jax.numpy as jnp
from jax.experimental import pallas as pl
from jax.experimental.pallas import tpu as pltpu


def _lookup_body(pe_ref, out_ref):
    out_ref[:, 0, :] = pe_ref[...]


def kernel(x, pos_embed):
    seq_len = x.shape[0]
    d_model = pos_embed.shape[1]
    return pl.pallas_call(
        _lookup_body,
        out_shape=jax.ShapeDtypeStruct((seq_len, 1, d_model), pos_embed.dtype),
        compiler_params=pltpu.CompilerParams(
            disable_bounds_checks=True,
            disable_semaphore_checks=True,
            skip_device_barrier=True,
        ),
    )(pos_embed[:seq_len])
